# baseline (speedup 1.0000x reference)
import functools

import jax
import jax.numpy as jnp
from jax import lax
from jax.experimental import pallas as pl
from jax.experimental.pallas import tpu as pltpu
from jax.experimental.pallas import tpu_sc as plsc

E = 64
NW = 32
CH = 128
K = 5
CHUNK = CH * K


def _gather_sc(table2, idx):
    N = idx.shape[0]
    b_per_w = N // NW
    n_chunks = b_per_w // CHUNK
    mesh = plsc.VectorSubcoreMesh(core_axis_name="c", subcore_axis_name="s")

    @functools.partial(
        pl.kernel,
        mesh=mesh,
        out_type=jax.ShapeDtypeStruct((N, 2 * E), jnp.float32),
        scratch_types=[
            pltpu.VMEM((CHUNK,), jnp.int32),
            pltpu.VMEM((CHUNK, 2 * E), jnp.float32),
            pltpu.SemaphoreType.DMA,
        ],
    )
    def k(table_hbm, idx_hbm, out_hbm, idx_v, rows_v, sem):
        c = lax.axis_index("c")
        s = lax.axis_index("s")
        wid = s * 2 + c
        base = wid * b_per_w

        def body(j, carry):
            off = base + j * CHUNK
            pltpu.sync_copy(idx_hbm.at[pl.ds(off, CHUNK)], idx_v)
            copies = []
            for t in range(K):
                copies.append(
                    pltpu.async_copy(
                        table_hbm.at[idx_v.at[pl.ds(t * CH, CH)]],
                        rows_v.at[pl.ds(t * CH, CH)],
                        sem,
                    )
                )
            for cp in copies:
                cp.wait()
            pltpu.sync_copy(rows_v, out_hbm.at[pl.ds(off, CHUNK)])
            return carry

        lax.fori_loop(0, n_chunks, body, 0)

    return k(table2, idx)


def _project_tc(x, p, w):
    N = x.shape[0]
    BLK = 2048
    grid = N // BLK

    def body(x_ref, p_ref, w_ref, o_ref):
        lo = x_ref[:, :E]
        hi = x_ref[:, E:]
        par = p_ref[...]
        sel = lo + par * (hi - lo)
        o_ref[...] = lax.dot_general(
            sel, w_ref[...], (((1,), (1,)), ((), ())),
            preferred_element_type=jnp.float32,
        )

    return pl.pallas_call(
        body,
        grid=(grid,),
        in_specs=[
            pl.BlockSpec((BLK, 2 * E), lambda i: (i, 0)),
            pl.BlockSpec((BLK, 1), lambda i: (i, 0)),
            pl.BlockSpec((E, E), lambda i: (0, 0)),
        ],
        out_specs=pl.BlockSpec((BLK, E), lambda i: (i, 0)),
        out_shape=jax.ShapeDtypeStruct((N, E), jnp.float32),
    )(x, p, w)


def kernel(sent1, sent2, embedding_table, W):
    B, S = sent1.shape
    n = B * S
    V = embedding_table.shape[0]
    table2 = embedding_table.reshape(V // 2, 2 * E)
    idx = jnp.concatenate(
        [sent1.reshape(-1), sent2.reshape(-1)]
    ).astype(jnp.int32)
    parity = (idx & 1).astype(jnp.float32).reshape(-1, 1)
    gathered = _gather_sc(table2, idx >> 1)
    y = _project_tc(gathered, parity, W)
    s1 = y[:n].reshape(B, S, E)
    s2 = y[n:].reshape(B, S, E)
    return (s1, s2)


# speedup vs baseline: 1.7126x; 1.7126x over previous
import functools

import jax
import jax.numpy as jnp
from jax import lax
from jax.experimental import pallas as pl
from jax.experimental.pallas import tpu as pltpu
from jax.experimental.pallas import tpu_sc as plsc

E = 64
NW = 32
CH = 128
K = 10
CHUNK = CH * K


def _gather_sc(table, idx1, idx2):
    N = idx1.shape[0]
    b_per_w = N // NW
    n_chunks = b_per_w // CHUNK
    mesh = plsc.VectorSubcoreMesh(core_axis_name="c", subcore_axis_name="s")

    @functools.partial(
        pl.kernel,
        mesh=mesh,
        out_type=(
            jax.ShapeDtypeStruct((N, E), jnp.float32),
            jax.ShapeDtypeStruct((N, E), jnp.float32),
        ),
        compiler_params=pltpu.CompilerParams(use_tc_tiling_on_sc=False),
        scratch_types=[
            pltpu.VMEM((CHUNK,), jnp.int32),
            pltpu.VMEM((CHUNK, E), jnp.float32),
            pltpu.SemaphoreType.DMA,
        ],
    )
    def k(table_hbm, idx1_hbm, idx2_hbm, out1_hbm, out2_hbm, idx_v, rows_v,
          sem):
        c = lax.axis_index("c")
        s = lax.axis_index("s")
        wid = s * 2 + c
        base = wid * b_per_w

        for idx_hbm, out_hbm in ((idx1_hbm, out1_hbm), (idx2_hbm, out2_hbm)):
            def body(j, carry, idx_hbm=idx_hbm, out_hbm=out_hbm):
                off = base + j * CHUNK
                pltpu.sync_copy(idx_hbm.at[pl.ds(off, CHUNK)], idx_v)
                copies = []
                for t in range(K):
                    copies.append(
                        pltpu.async_copy(
                            table_hbm.at[idx_v.at[pl.ds(t * CH, CH)]],
                            rows_v.at[pl.ds(t * CH, CH)],
                            sem,
                        )
                    )
                for cp in copies:
                    cp.wait()
                pltpu.sync_copy(rows_v, out_hbm.at[pl.ds(off, CHUNK)])
                return carry

            lax.fori_loop(0, n_chunks, body, 0)

    return k(table, idx1, idx2)


def _project_tc(x3, w, S, B):
    def body(x_ref, w_ref, o_ref):
        o_ref[0] = lax.dot_general(
            w_ref[...], x_ref[0], (((1,), (1,)), ((), ())),
            preferred_element_type=jnp.float32,
        )

    return pl.pallas_call(
        body,
        grid=(S,),
        in_specs=[
            pl.BlockSpec((1, B, E), lambda s: (s, 0, 0)),
            pl.BlockSpec((E, E), lambda s: (0, 0)),
        ],
        out_specs=pl.BlockSpec((1, E, B), lambda s: (s, 0, 0)),
        out_shape=jax.ShapeDtypeStruct((S, E, B), jnp.float32),
    )(x3, w)


def kernel(sent1, sent2, embedding_table, W):
    B, S = sent1.shape
    idx1 = sent1.T.reshape(-1).astype(jnp.int32)
    idx2 = sent2.T.reshape(-1).astype(jnp.int32)
    g1, g2 = _gather_sc(embedding_table, idx1, idx2)
    y1 = _project_tc(g1.reshape(S, B, E), W, S, B)
    y2 = _project_tc(g2.reshape(S, B, E), W, S, B)
    s1 = y1.transpose(2, 0, 1)
    s2 = y2.transpose(2, 0, 1)
    return (s1, s2)


# speedup vs baseline: 1.8513x; 1.0810x over previous
import functools

import jax
import jax.numpy as jnp
from jax import lax
from jax.experimental import pallas as pl
from jax.experimental.pallas import tpu as pltpu
from jax.experimental.pallas import tpu_sc as plsc

E = 64
NW = 32
CH = 128
K = 5
CHUNK = CH * K


def _gather_sc(table2, idx1, idx2):
    N = idx1.shape[0]
    b_per_w = N // NW
    n_chunks = b_per_w // CHUNK
    mesh = plsc.VectorSubcoreMesh(core_axis_name="c", subcore_axis_name="s")

    @functools.partial(
        pl.kernel,
        mesh=mesh,
        out_type=(
            jax.ShapeDtypeStruct((N, 2 * E), jnp.float32),
            jax.ShapeDtypeStruct((N, 2 * E), jnp.float32),
        ),
        scratch_types=[
            pltpu.VMEM((CHUNK,), jnp.int32),
            pltpu.VMEM((CHUNK, 2 * E), jnp.float32),
            pltpu.SemaphoreType.DMA,
        ],
    )
    def k(table_hbm, idx1_hbm, idx2_hbm, out1_hbm, out2_hbm, idx_v, rows_v,
          sem):
        c = lax.axis_index("c")
        s = lax.axis_index("s")
        wid = s * 2 + c
        base = wid * b_per_w

        for idx_hbm, out_hbm in ((idx1_hbm, out1_hbm), (idx2_hbm, out2_hbm)):
            def body(j, carry, idx_hbm=idx_hbm, out_hbm=out_hbm):
                off = base + j * CHUNK
                pltpu.sync_copy(idx_hbm.at[pl.ds(off, CHUNK)], idx_v)
                copies = []
                for t in range(K):
                    copies.append(
                        pltpu.async_copy(
                            table_hbm.at[idx_v.at[pl.ds(t * CH, CH)]],
                            rows_v.at[pl.ds(t * CH, CH)],
                            sem,
                        )
                    )
                for cp in copies:
                    cp.wait()
                pltpu.sync_copy(rows_v, out_hbm.at[pl.ds(off, CHUNK)])
                return carry

            lax.fori_loop(0, n_chunks, body, 0)

    return k(table2, idx1, idx2)


def _project_tc(x3, w2, par, S, B):
    def body(x_ref, w_ref, p_ref, o_ref):
        z = lax.dot_general(
            w_ref[...], x_ref[0], (((1,), (1,)), ((), ())),
            preferred_element_type=jnp.float32,
        )
        zlo = z[:E, :]
        zhi = z[E:, :]
        o_ref[0] = zlo + p_ref[0] * (zhi - zlo)

    return pl.pallas_call(
        body,
        grid=(S,),
        in_specs=[
            pl.BlockSpec((1, B, 2 * E), lambda s: (s, 0, 0)),
            pl.BlockSpec((2 * E, 2 * E), lambda s: (0, 0)),
            pl.BlockSpec((1, 1, B), lambda s: (s, 0, 0)),
        ],
        out_specs=pl.BlockSpec((1, E, B), lambda s: (s, 0, 0)),
        out_shape=jax.ShapeDtypeStruct((S, E, B), jnp.float32),
    )(x3, w2, par)


def kernel(sent1, sent2, embedding_table, W):
    B, S = sent1.shape
    V = embedding_table.shape[0]
    table2 = embedding_table.reshape(V // 2, 2 * E)
    i1 = sent1.T.astype(jnp.int32)
    i2 = sent2.T.astype(jnp.int32)
    g1, g2 = _gather_sc(table2, (i1 >> 1).reshape(-1), (i2 >> 1).reshape(-1))
    zero = jnp.zeros((E, E), jnp.float32)
    w2 = jnp.block([[W, zero], [zero, W]])
    p1 = (i1 & 1).astype(jnp.float32).reshape(S, 1, B)
    p2 = (i2 & 1).astype(jnp.float32).reshape(S, 1, B)
    y1 = _project_tc(g1.reshape(S, B, 2 * E), w2, p1, S, B)
    y2 = _project_tc(g2.reshape(S, B, 2 * E), w2, p2, S, B)
    s1 = y1.transpose(2, 0, 1)
    s2 = y2.transpose(2, 0, 1)
    return (s1, s2)


# speedup vs baseline: 2.2045x; 1.1908x over previous
import functools

import jax
import jax.numpy as jnp
from jax import lax
from jax.experimental import pallas as pl
from jax.experimental.pallas import tpu as pltpu
from jax.experimental.pallas import tpu_sc as plsc

E = 64
NW = 32
CH = 128
K = 5
CHUNK = CH * K
VB = 2048


def _pack_table_tc(table_t):
    V = table_t.shape[1]
    grid = (V + VB - 1) // VB
    H = VB // 2

    def body(x_ref, o_ref):
        x = x_ref[...]
        o_ref[...] = jnp.concatenate(
            [x[:, :H].T, x[:, H:].T], axis=1)

    return pl.pallas_call(
        body,
        grid=(grid,),
        in_specs=[pl.BlockSpec((E, VB), lambda j: (0, j))],
        out_specs=pl.BlockSpec((H, 2 * E), lambda j: (j, 0)),
        out_shape=jax.ShapeDtypeStruct((grid * H, 2 * E), jnp.float32),
    )(table_t)


def _gather_sc(table2, idx1, idx2):
    N = idx1.shape[0]
    b_per_w = N // NW
    n_chunks = b_per_w // CHUNK
    mesh = plsc.VectorSubcoreMesh(core_axis_name="c", subcore_axis_name="s")

    @functools.partial(
        pl.kernel,
        mesh=mesh,
        out_type=(
            jax.ShapeDtypeStruct((N, 2 * E), jnp.float32),
            jax.ShapeDtypeStruct((N, 2 * E), jnp.float32),
        ),
        scratch_types=[
            pltpu.VMEM((CHUNK,), jnp.int32),
            pltpu.VMEM((CHUNK, 2 * E), jnp.float32),
            pltpu.SemaphoreType.DMA,
        ],
    )
    def k(table_hbm, idx1_hbm, idx2_hbm, out1_hbm, out2_hbm, idx_v, rows_v,
          sem):
        c = lax.axis_index("c")
        s = lax.axis_index("s")
        wid = s * 2 + c
        base = wid * b_per_w

        for idx_hbm, out_hbm in ((idx1_hbm, out1_hbm), (idx2_hbm, out2_hbm)):
            def body(j, carry, idx_hbm=idx_hbm, out_hbm=out_hbm):
                off = base + j * CHUNK
                pltpu.sync_copy(idx_hbm.at[pl.ds(off, CHUNK)], idx_v)
                copies = []
                for t in range(K):
                    copies.append(
                        pltpu.async_copy(
                            table_hbm.at[idx_v.at[pl.ds(t * CH, CH)]],
                            rows_v.at[pl.ds(t * CH, CH)],
                            sem,
                        )
                    )
                for cp in copies:
                    cp.wait()
                pltpu.sync_copy(rows_v, out_hbm.at[pl.ds(off, CHUNK)])
                return carry

            lax.fori_loop(0, n_chunks, body, 0)

    return k(table2, idx1, idx2)


def _project_tc(x3, w2, par, S, B):
    def body(x_ref, w_ref, p_ref, o_ref):
        z = lax.dot_general(
            w_ref[...], x_ref[0], (((1,), (1,)), ((), ())),
            preferred_element_type=jnp.float32,
        )
        zlo = z[:E, :]
        zhi = z[E:, :]
        o_ref[0] = zlo + p_ref[0] * (zhi - zlo)

    return pl.pallas_call(
        body,
        grid=(S,),
        in_specs=[
            pl.BlockSpec((1, B, 2 * E), lambda s: (s, 0, 0)),
            pl.BlockSpec((2 * E, 2 * E), lambda s: (0, 0)),
            pl.BlockSpec((1, 1, B), lambda s: (s, 0, 0)),
        ],
        out_specs=pl.BlockSpec((1, E, B), lambda s: (s, 0, 0)),
        out_shape=jax.ShapeDtypeStruct((S, E, B), jnp.float32),
    )(x3, w2, par)


def kernel(sent1, sent2, embedding_table, W):
    B, S = sent1.shape
    H = VB // 2
    table2 = _pack_table_tc(embedding_table.T)
    i1 = sent1.T.astype(jnp.int32)
    i2 = sent2.T.astype(jnp.int32)
    def pack_idx(i):
        blk = i >> 11
        o = i & (VB - 1)
        return (blk << 10) | (o & (H - 1)), (o >> 10) & 1

    r1, h1 = pack_idx(i1)
    r2, h2 = pack_idx(i2)
    g1, g2 = _gather_sc(table2, r1.reshape(-1), r2.reshape(-1))
    zero = jnp.zeros((E, E), jnp.float32)
    w2 = jnp.block([[W, zero], [zero, W]])
    p1 = h1.astype(jnp.float32).reshape(S, 1, B)
    p2 = h2.astype(jnp.float32).reshape(S, 1, B)
    y1 = _project_tc(g1.reshape(S, B, 2 * E), w2, p1, S, B)
    y2 = _project_tc(g2.reshape(S, B, 2 * E), w2, p2, S, B)
    s1 = y1.transpose(2, 0, 1)
    s2 = y2.transpose(2, 0, 1)
    return (s1, s2)
